# baseline (speedup 1.0000x reference)
import functools

import jax
import jax.numpy as jnp
import numpy as np
from jax import lax
from jax.experimental import pallas as pl
from jax.experimental.pallas import tpu as pltpu
from jax.experimental.pallas import tpu_sc as plsc

NC, NS, L = 2, 16, 16
NW = NC * NS
NT = 8192
NCOL = 1536
CPW = NCOL // NW
CHUNK = NT // L
NVEC = NT // L
RADIX = 256
HIST = RADIX * L
MININT = np.int32(-2147483648)


def _transpose_body(x_ref, o_ref):
    o_ref[...] = lax.bitcast_convert_type(x_ref[...].T, jnp.int32)


def _transpose(x):
    bt, bc = 1024, 512
    return pl.pallas_call(
        _transpose_body,
        grid=(NT // bt, NCOL // bc),
        in_specs=[pl.BlockSpec((bt, bc), lambda i, j: (i, j))],
        out_specs=pl.BlockSpec((bc, bt), lambda i, j: (j, i)),
        out_shape=jax.ShapeDtypeStruct((NCOL, NT), jnp.int32),
    )(x)


def _sc_body(pred_hbm, obs_hbm, out_hbm,
             a0, a1, a2, a3, b0, b1, b2, b3,
             hx0, hx1, hx2, hx3, hy0, hy1, hy2, hy3,
             tt0, tt1, tt2, tt3, dg0, dg1, dg2, dg3,
             stage, sem):
    cid = lax.axis_index("c")
    sid = lax.axis_index("s")
    wid = sid * NC + cid
    lane = lax.iota(jnp.int32, L)
    ones = jnp.ones((L,), jnp.int32)
    zi = jnp.zeros((L,), jnp.int32)
    fz = jnp.zeros((L,), jnp.float32)
    lane0 = lane == 0
    A = [a0, a1, a2, a3]
    B = [b0, b1, b2, b3]
    HX = [hx0, hx1, hx2, hx3]
    HY = [hy0, hy1, hy2, hy3]
    TOT = [tt0, tt1, tt2, tt3]
    DB = [dg0, dg1, dg2, dg3]

    def dmad(s, c0):
        ref = pred_hbm if s % 2 == 0 else obs_hbm
        return pltpu.make_async_copy(ref.at[c0 + s // 2], A[s], sem)

    def zinit(i, _):
        for s in range(4):
            HX[s][pl.ds(i * L, L)] = zi
        return 0

    lax.fori_loop(0, HIST // L, zinit, 0)

    base0 = wid * CPW
    for s in range(4):
        dmad(s, base0).start()

    def half_step(c0, cnext, HA, HB, accs):
        with jax.named_scope("dmawait"):
            for s in range(4):
                dmad(s, c0).wait()

        with jax.named_scope("h0"):
            @plsc.parallel_loop(0, NVEC, unroll=2)
            def h0(i):
                ys = [A[s][pl.ds(i * L, L)] for s in range(4)]
                for s in range(4):
                    m = ys[s] ^ (MININT | (ys[s] >> 31))
                    d = lax.shift_right_logical(m, 8) & 255
                    plsc.addupdate_scatter(HA[s], [(d << 4) | lane], ones)

        for t in range(3):
            cur = HA if t % 2 == 0 else HB
            nxt = HB if t % 2 == 0 else HA
            src = A if t % 2 == 0 else B
            dst = B if t % 2 == 0 else A

            with jax.named_scope("scan1"):
                @plsc.parallel_loop(0, RADIX, unroll=2)
                def l1(g):
                    vs = [cur[s][pl.ds(g * L, L)] for s in range(4)]
                    for s in range(4):
                        sc = plsc.cumsum(vs[s])
                        cur[s][pl.ds(g * L, L)] = sc - vs[s]
                        plsc.store_scatter(TOT[s], [g + zi], jnp.sum(vs[s]) + zi,
                                           mask=lane0)

            def l2(g, carry):
                nc = []
                for s in range(4):
                    v = TOT[s][pl.ds(g * L, L)]
                    sc = plsc.cumsum(v)
                    DB[s][pl.ds(g * L, L)] = (sc - v) + carry[s]
                    nc.append(carry[s] + jnp.sum(v))
                return tuple(nc)

            with jax.named_scope("scan2"):
                plsc.parallel_loop(0, RADIX // L, carry=(jnp.int32(0),) * 4)(l2)

            with jax.named_scope("scan3"):
                @plsc.parallel_loop(0, RADIX // L)
                def l3(gg):
                    dvecs = [DB[s][pl.ds(gg * L, L)] for s in range(4)]
                    for j in range(L):
                        exls = [cur[s][pl.ds(gg * RADIX + j * L, L)] for s in range(4)]
                        for s in range(4):
                            cur[s][pl.ds(gg * RADIX + j * L, L)] = exls[s] + dvecs[s][j]
                            nxt[s][pl.ds(gg * RADIX + j * L, L)] = zi

            def pb(i, _):
                ks = [src[s][pl.ds(i * L, L)] for s in range(4)]
                if t == 0:
                    outs = [k ^ (MININT | (k >> 31)) for k in ks]
                    digs = [lax.shift_right_logical(m, 8) & 255 for m in outs]
                elif t == 1:
                    outs = ks
                    digs = [lax.shift_right_logical(k, 16) & 255 for k in ks]
                else:
                    outs = ks
                    digs = [lax.shift_right_logical(k, 24) for k in ks]
                hidxs = [(d << 4) | lane for d in digs]
                rs = [plsc.load_gather(cur[s], [hidxs[s]]) for s in range(4)]
                for s in range(4):
                    plsc.store_scatter(cur[s], [hidxs[s]], rs[s] + ones)
                for s in range(4):
                    pos = ((rs[s] & (CHUNK - 1)) << 4) | (rs[s] >> 9)
                    plsc.store_scatter(dst[s], [pos], outs[s])
                return 0

            with jax.named_scope("permute"):
                lax.fori_loop(0, NVEC, pb, 0, unroll=4)

            if t < 2:
                sh2 = 16 if t == 0 else 24

                with jax.named_scope("hnx"):
                    @plsc.parallel_loop(0, NVEC, unroll=2)
                    def hnx(i):
                        ks = [dst[s][pl.ds(i * L, L)] for s in range(4)]
                        for s in range(4):
                            d = lax.shift_right_logical(ks[s], sh2)
                            if sh2 < 24:
                                d = d & 255
                            plsc.addupdate_scatter(nxt[s], [(d << 4) | lane], ones)

        for s in range(4):
            dmad(s, cnext).start()

        def dbody(i, acc):
            ys = [B[s][pl.ds((2 * i + par) * L, L)]
                  for par in range(2) for s in range(4)]
            fs = []
            for y in ys:
                x = y ^ (MININT | ~(y >> 31))
                fs.append(lax.bitcast_convert_type(x, jnp.float32))
            na = list(acc)
            for par in range(2):
                for pi in range(2):
                    fa = fs[par * 4 + 2 * pi]
                    fb = fs[par * 4 + 2 * pi + 1]
                    na[2 * pi + par] = na[2 * pi + par] + jnp.abs(fa - fb)
            return tuple(na)

        with jax.named_scope("diff"):
            return plsc.parallel_loop(0, NVEC // 2, carry=accs)(dbody)

    def outer(q, accs):
        c0 = base0 + 4 * q
        accs = half_step(c0, c0 + 2, HX, HY, accs)
        cnext = jnp.minimum(c0 + 4, NCOL - 2)
        accs = half_step(c0 + 2, cnext, HY, HX, accs)
        return accs

    accs = lax.fori_loop(0, CPW // 4, outer, (fz, fz, fz, fz))
    for s in range(4):
        dmad(s, 0).wait()
    stage[...] = (accs[0] + accs[1]) + (accs[2] + accs[3])
    pltpu.sync_copy(stage, out_hbm.at[wid])


_sc_wasserstein = functools.partial(
    pl.kernel,
    out_type=jax.ShapeDtypeStruct((NW, L), jnp.float32),
    mesh=plsc.VectorSubcoreMesh(core_axis_name="c", subcore_axis_name="s"),
    compiler_params=pltpu.CompilerParams(needs_layout_passes=False),
    scratch_types=(
        [pltpu.VMEM((NT,), jnp.int32) for _ in range(8)]
        + [pltpu.VMEM((HIST,), jnp.int32) for _ in range(8)]
        + [pltpu.VMEM((RADIX,), jnp.int32) for _ in range(8)]
        + [pltpu.VMEM((L,), jnp.float32)]
        + [pltpu.SemaphoreType.DMA]
    ),
)(_sc_body)


def kernel(pred_waveforms, obs_waveforms):
    nt, ntr, ch = pred_waveforms.shape
    pred_t = _transpose(pred_waveforms.reshape(nt, ntr * ch))
    obs_t = _transpose(obs_waveforms.reshape(nt, ntr * ch))
    partials = _sc_wasserstein(pred_t, obs_t)
    return jnp.sum(partials) / (nt * ntr * ch)


# speedup vs baseline: 1.1873x; 1.1873x over previous
import functools

import jax
import jax.numpy as jnp
import numpy as np
from jax import lax
from jax.experimental import pallas as pl
from jax.experimental.pallas import tpu as pltpu
from jax.experimental.pallas import tpu_sc as plsc

NC, NS, L = 2, 16, 16
NW = NC * NS
NT = 8192
NCOL = 1536
CPW = NCOL // NW
CHUNK = NT // L
NVEC = NT // L
RADIX = 256
HIST = RADIX * L
MININT = np.int32(-2147483648)


def _transpose_body(x_ref, o_ref):
    o_ref[...] = lax.bitcast_convert_type(x_ref[...].T, jnp.int32)


def _transpose(x):
    bt, bc = 1024, 512
    return pl.pallas_call(
        _transpose_body,
        grid=(NT // bt, NCOL // bc),
        in_specs=[pl.BlockSpec((bt, bc), lambda i, j: (i, j))],
        out_specs=pl.BlockSpec((bc, bt), lambda i, j: (j, i)),
        out_shape=jax.ShapeDtypeStruct((NCOL, NT), jnp.int32),
    )(x)


def _sc_body(pred_hbm, obs_hbm, out_hbm,
             a0, a1, a2, a3, b0, b1, b2, b3,
             hx0, hx1, hx2, hx3, hy0, hy1, hy2, hy3,
             tt0, tt1, tt2, tt3, dg0, dg1, dg2, dg3,
             stage, sem):
    cid = lax.axis_index("c")
    sid = lax.axis_index("s")
    wid = sid * NC + cid
    lane = lax.iota(jnp.int32, L)
    ones = jnp.ones((L,), jnp.int32)
    zi = jnp.zeros((L,), jnp.int32)
    fz = jnp.zeros((L,), jnp.float32)
    lane0 = lane == 0
    A = [a0, a1, a2, a3]
    B = [b0, b1, b2, b3]
    HX = [hx0, hx1, hx2, hx3]
    HY = [hy0, hy1, hy2, hy3]
    TOT = [tt0, tt1, tt2, tt3]
    DB = [dg0, dg1, dg2, dg3]

    def dmad(s, c0):
        ref = pred_hbm if s % 2 == 0 else obs_hbm
        return pltpu.make_async_copy(ref.at[c0 + s // 2], A[s], sem)

    def zinit(i, _):
        for s in range(4):
            HX[s][pl.ds(i * L, L)] = zi
        return 0

    lax.fori_loop(0, HIST // L, zinit, 0)

    base0 = wid * CPW
    for s in range(4):
        dmad(s, base0).start()

    def half_step(c0, cnext, HA, HB, accs):
        with jax.named_scope("dmawait"):
            for s in range(4):
                dmad(s, c0).wait()

        with jax.named_scope("h0"):
            @plsc.parallel_loop(0, NVEC, unroll=2)
            def h0(i):
                ys = [A[s][pl.ds(i * L, L)] for s in range(4)]
                for s in range(4):
                    m = ys[s] ^ (MININT | (ys[s] >> 31))
                    d = lax.shift_right_logical(m, 8) & 255
                    plsc.addupdate_scatter(HA[s], [(d << 4) | lane], ones)

        for t in range(3):
            cur = HA if t % 2 == 0 else HB
            nxt = HB if t % 2 == 0 else HA
            src = A if t % 2 == 0 else B
            dst = B if t % 2 == 0 else A

            with jax.named_scope("scan1"):
                @plsc.parallel_loop(0, RADIX, unroll=2)
                def l1(g):
                    vs = [cur[s][pl.ds(g * L, L)] for s in range(4)]
                    for s in range(4):
                        sc = plsc.cumsum(vs[s])
                        cur[s][pl.ds(g * L, L)] = sc - vs[s]
                        plsc.store_scatter(TOT[s], [g + zi], jnp.sum(vs[s]) + zi,
                                           mask=lane0)

            def l2(g, carry):
                nc = []
                for s in range(4):
                    v = TOT[s][pl.ds(g * L, L)]
                    sc = plsc.cumsum(v)
                    DB[s][pl.ds(g * L, L)] = (sc - v) + carry[s]
                    nc.append(carry[s] + jnp.sum(v))
                return tuple(nc)

            with jax.named_scope("scan2"):
                plsc.parallel_loop(0, RADIX // L, carry=(jnp.int32(0),) * 4)(l2)

            with jax.named_scope("scan3"):
                @plsc.parallel_loop(0, RADIX // L)
                def l3(gg):
                    dvecs = [DB[s][pl.ds(gg * L, L)] for s in range(4)]
                    for j in range(L):
                        exls = [cur[s][pl.ds(gg * RADIX + j * L, L)] for s in range(4)]
                        for s in range(4):
                            cur[s][pl.ds(gg * RADIX + j * L, L)] = exls[s] + dvecs[s][j]
                            nxt[s][pl.ds(gg * RADIX + j * L, L)] = zi

            def keys_at(i):
                ks = [src[s][pl.ds(i * L, L)] for s in range(4)]
                if t == 0:
                    outs = [k ^ (MININT | (k >> 31)) for k in ks]
                    digs = [lax.shift_right_logical(m, 8) & 255 for m in outs]
                elif t == 1:
                    outs = ks
                    digs = [lax.shift_right_logical(k, 16) & 255 for k in ks]
                else:
                    outs = ks
                    digs = [lax.shift_right_logical(k, 24) for k in ks]
                return tuple(outs), tuple([(d << 4) | lane for d in digs])

            def pb(i, carry):
                outs, hidxs = carry
                rs = [plsc.load_gather(cur[s], [hidxs[s]]) for s in range(4)]
                nxt_carry = keys_at(jnp.minimum(i + 1, NVEC - 1))
                for s in range(4):
                    plsc.store_scatter(cur[s], [hidxs[s]], rs[s] + ones)
                for s in range(4):
                    pos = ((rs[s] & (CHUNK - 1)) << 4) | (rs[s] >> 9)
                    plsc.store_scatter(dst[s], [pos], outs[s])
                return nxt_carry

            with jax.named_scope("permute"):
                lax.fori_loop(0, NVEC, pb, keys_at(0), unroll=4)

            if t < 2:
                sh2 = 16 if t == 0 else 24

                with jax.named_scope("hnx"):
                    @plsc.parallel_loop(0, NVEC, unroll=2)
                    def hnx(i):
                        ks = [dst[s][pl.ds(i * L, L)] for s in range(4)]
                        for s in range(4):
                            d = lax.shift_right_logical(ks[s], sh2)
                            if sh2 < 24:
                                d = d & 255
                            plsc.addupdate_scatter(nxt[s], [(d << 4) | lane], ones)

        for s in range(4):
            dmad(s, cnext).start()

        def dbody(i, acc):
            ys = [B[s][pl.ds((2 * i + par) * L, L)]
                  for par in range(2) for s in range(4)]
            fs = []
            for y in ys:
                x = y ^ (MININT | ~(y >> 31))
                fs.append(lax.bitcast_convert_type(x, jnp.float32))
            na = list(acc)
            for par in range(2):
                for pi in range(2):
                    fa = fs[par * 4 + 2 * pi]
                    fb = fs[par * 4 + 2 * pi + 1]
                    na[2 * pi + par] = na[2 * pi + par] + jnp.abs(fa - fb)
            return tuple(na)

        with jax.named_scope("diff"):
            return plsc.parallel_loop(0, NVEC // 2, carry=accs)(dbody)

    def outer(q, accs):
        c0 = base0 + 4 * q
        accs = half_step(c0, c0 + 2, HX, HY, accs)
        cnext = jnp.minimum(c0 + 4, NCOL - 2)
        accs = half_step(c0 + 2, cnext, HY, HX, accs)
        return accs

    accs = lax.fori_loop(0, CPW // 4, outer, (fz, fz, fz, fz))
    for s in range(4):
        dmad(s, 0).wait()
    stage[...] = (accs[0] + accs[1]) + (accs[2] + accs[3])
    pltpu.sync_copy(stage, out_hbm.at[wid])


_sc_wasserstein = functools.partial(
    pl.kernel,
    out_type=jax.ShapeDtypeStruct((NW, L), jnp.float32),
    mesh=plsc.VectorSubcoreMesh(core_axis_name="c", subcore_axis_name="s"),
    compiler_params=pltpu.CompilerParams(needs_layout_passes=False),
    scratch_types=(
        [pltpu.VMEM((NT,), jnp.int32) for _ in range(8)]
        + [pltpu.VMEM((HIST,), jnp.int32) for _ in range(8)]
        + [pltpu.VMEM((RADIX,), jnp.int32) for _ in range(8)]
        + [pltpu.VMEM((L,), jnp.float32)]
        + [pltpu.SemaphoreType.DMA]
    ),
)(_sc_body)


def kernel(pred_waveforms, obs_waveforms):
    nt, ntr, ch = pred_waveforms.shape
    pred_t = _transpose(pred_waveforms.reshape(nt, ntr * ch))
    obs_t = _transpose(obs_waveforms.reshape(nt, ntr * ch))
    partials = _sc_wasserstein(pred_t, obs_t)
    return jnp.sum(partials) / (nt * ntr * ch)


# speedup vs baseline: 1.1960x; 1.0073x over previous
import functools

import jax
import jax.numpy as jnp
import numpy as np
from jax import lax
from jax.experimental import pallas as pl
from jax.experimental.pallas import tpu as pltpu
from jax.experimental.pallas import tpu_sc as plsc

NC, NS, L = 2, 16, 16
NW = NC * NS
NT = 8192
NCOL = 1536
CPW = NCOL // NW
CHUNK = NT // L
NVEC = NT // L
RADIX = 256
HIST = RADIX * L
MININT = np.int32(-2147483648)


def _transpose_body(x_ref, o_ref):
    o_ref[...] = lax.bitcast_convert_type(x_ref[...].T, jnp.int32)


def _transpose(x):
    bt, bc = 1024, 512
    return pl.pallas_call(
        _transpose_body,
        grid=(NT // bt, NCOL // bc),
        in_specs=[pl.BlockSpec((bt, bc), lambda i, j: (i, j))],
        out_specs=pl.BlockSpec((bc, bt), lambda i, j: (j, i)),
        out_shape=jax.ShapeDtypeStruct((NCOL, NT), jnp.int32),
    )(x)


def _sc_body(pred_hbm, obs_hbm, out_hbm,
             a0, a1, a2, a3, b0, b1, b2, b3,
             hx0, hx1, hx2, hx3, hy0, hy1, hy2, hy3,
             tt0, tt1, tt2, tt3, dg0, dg1, dg2, dg3,
             stage, sem):
    cid = lax.axis_index("c")
    sid = lax.axis_index("s")
    wid = sid * NC + cid
    lane = lax.iota(jnp.int32, L)
    ones = jnp.ones((L,), jnp.int32)
    zi = jnp.zeros((L,), jnp.int32)
    fz = jnp.zeros((L,), jnp.float32)
    lane0 = lane == 0
    A = [a0, a1, a2, a3]
    B = [b0, b1, b2, b3]
    HX = [hx0, hx1, hx2, hx3]
    HY = [hy0, hy1, hy2, hy3]
    TOT = [tt0, tt1, tt2, tt3]
    DB = [dg0, dg1, dg2, dg3]

    def dmad(s, c0):
        ref = pred_hbm if s % 2 == 0 else obs_hbm
        return pltpu.make_async_copy(ref.at[c0 + s // 2], A[s], sem)

    def zinit(i, _):
        for s in range(4):
            HX[s][pl.ds(i * L, L)] = zi
        return 0

    lax.fori_loop(0, HIST // L, zinit, 0)

    base0 = wid * CPW
    for s in range(4):
        dmad(s, base0).start()

    def half_step(c0, cnext, HA, HB, accs):
        with jax.named_scope("dmawait"):
            for s in range(4):
                dmad(s, c0).wait()

        with jax.named_scope("h0"):
            @plsc.parallel_loop(0, NVEC, unroll=4)
            def h0(i):
                ys = [A[s][pl.ds(i * L, L)] for s in range(4)]
                for s in range(4):
                    m = ys[s] ^ (MININT | (ys[s] >> 31))
                    d = lax.shift_right_logical(m, 8) & 255
                    plsc.addupdate_scatter(HA[s], [(d << 4) | lane], ones)

        for t in range(3):
            cur = HA if t % 2 == 0 else HB
            nxt = HB if t % 2 == 0 else HA
            src = A if t % 2 == 0 else B
            dst = B if t % 2 == 0 else A

            with jax.named_scope("scan1"):
                @plsc.parallel_loop(0, RADIX, unroll=4)
                def l1(g):
                    vs = [cur[s][pl.ds(g * L, L)] for s in range(4)]
                    for s in range(4):
                        sc = plsc.cumsum(vs[s])
                        cur[s][pl.ds(g * L, L)] = sc - vs[s]
                        plsc.store_scatter(TOT[s], [g + zi], jnp.sum(vs[s]) + zi,
                                           mask=lane0)

            def l2(g, carry):
                nc = []
                for s in range(4):
                    v = TOT[s][pl.ds(g * L, L)]
                    sc = plsc.cumsum(v)
                    DB[s][pl.ds(g * L, L)] = (sc - v) + carry[s]
                    nc.append(carry[s] + jnp.sum(v))
                return tuple(nc)

            with jax.named_scope("scan2"):
                plsc.parallel_loop(0, RADIX // L, carry=(jnp.int32(0),) * 4)(l2)

            with jax.named_scope("scan3"):
                @plsc.parallel_loop(0, RADIX // L)
                def l3(gg):
                    dvecs = [DB[s][pl.ds(gg * L, L)] for s in range(4)]
                    for j in range(L):
                        exls = [cur[s][pl.ds(gg * RADIX + j * L, L)] for s in range(4)]
                        for s in range(4):
                            cur[s][pl.ds(gg * RADIX + j * L, L)] = exls[s] + dvecs[s][j]
                            nxt[s][pl.ds(gg * RADIX + j * L, L)] = zi

            def keys_at(i):
                ks = [src[s][pl.ds(i * L, L)] for s in range(4)]
                if t == 0:
                    outs = [k ^ (MININT | (k >> 31)) for k in ks]
                    digs = [lax.shift_right_logical(m, 8) & 255 for m in outs]
                elif t == 1:
                    outs = ks
                    digs = [lax.shift_right_logical(k, 16) & 255 for k in ks]
                else:
                    outs = ks
                    digs = [lax.shift_right_logical(k, 24) for k in ks]
                return tuple(outs), tuple([(d << 4) | lane for d in digs])

            def pb(i, carry):
                outs, hidxs = carry
                rs = [plsc.load_gather(cur[s], [hidxs[s]]) for s in range(4)]
                nxt_carry = keys_at(jnp.minimum(i + 1, NVEC - 1))
                for s in range(4):
                    plsc.store_scatter(cur[s], [hidxs[s]], rs[s] + ones)
                for s in range(4):
                    pos = ((rs[s] & (CHUNK - 1)) << 4) | (rs[s] >> 9)
                    plsc.store_scatter(dst[s], [pos], outs[s])
                return nxt_carry

            with jax.named_scope("permute"):
                lax.fori_loop(0, NVEC, pb, keys_at(0), unroll=8)

            if t < 2:
                sh2 = 16 if t == 0 else 24

                with jax.named_scope("hnx"):
                    @plsc.parallel_loop(0, NVEC, unroll=4)
                    def hnx(i):
                        ks = [dst[s][pl.ds(i * L, L)] for s in range(4)]
                        for s in range(4):
                            d = lax.shift_right_logical(ks[s], sh2)
                            if sh2 < 24:
                                d = d & 255
                            plsc.addupdate_scatter(nxt[s], [(d << 4) | lane], ones)

        for s in range(4):
            dmad(s, cnext).start()

        def dbody(i, acc):
            ys = [B[s][pl.ds((2 * i + par) * L, L)]
                  for par in range(2) for s in range(4)]
            fs = []
            for y in ys:
                x = y ^ (MININT | ~(y >> 31))
                fs.append(lax.bitcast_convert_type(x, jnp.float32))
            na = list(acc)
            for par in range(2):
                for pi in range(2):
                    fa = fs[par * 4 + 2 * pi]
                    fb = fs[par * 4 + 2 * pi + 1]
                    na[2 * pi + par] = na[2 * pi + par] + jnp.abs(fa - fb)
            return tuple(na)

        with jax.named_scope("diff"):
            return plsc.parallel_loop(0, NVEC // 2, carry=accs)(dbody)

    def outer(q, accs):
        c0 = base0 + 4 * q
        accs = half_step(c0, c0 + 2, HX, HY, accs)
        cnext = jnp.minimum(c0 + 4, NCOL - 2)
        accs = half_step(c0 + 2, cnext, HY, HX, accs)
        return accs

    accs = lax.fori_loop(0, CPW // 4, outer, (fz, fz, fz, fz))
    for s in range(4):
        dmad(s, 0).wait()
    stage[...] = (accs[0] + accs[1]) + (accs[2] + accs[3])
    pltpu.sync_copy(stage, out_hbm.at[wid])


_sc_wasserstein = functools.partial(
    pl.kernel,
    out_type=jax.ShapeDtypeStruct((NW, L), jnp.float32),
    mesh=plsc.VectorSubcoreMesh(core_axis_name="c", subcore_axis_name="s"),
    compiler_params=pltpu.CompilerParams(needs_layout_passes=False),
    scratch_types=(
        [pltpu.VMEM((NT,), jnp.int32) for _ in range(8)]
        + [pltpu.VMEM((HIST,), jnp.int32) for _ in range(8)]
        + [pltpu.VMEM((RADIX,), jnp.int32) for _ in range(8)]
        + [pltpu.VMEM((L,), jnp.float32)]
        + [pltpu.SemaphoreType.DMA]
    ),
)(_sc_body)


def kernel(pred_waveforms, obs_waveforms):
    nt, ntr, ch = pred_waveforms.shape
    pred_t = _transpose(pred_waveforms.reshape(nt, ntr * ch))
    obs_t = _transpose(obs_waveforms.reshape(nt, ntr * ch))
    partials = _sc_wasserstein(pred_t, obs_t)
    return jnp.sum(partials) / (nt * ntr * ch)


# speedup vs baseline: 1.3161x; 1.1004x over previous
import functools

import jax
import jax.numpy as jnp
import numpy as np
from jax import lax
from jax.experimental import pallas as pl
from jax.experimental.pallas import tpu as pltpu
from jax.experimental.pallas import tpu_sc as plsc

NC, NS, L = 2, 16, 16
NW = NC * NS
NT = 8192
NCOL = 1536
CPW = NCOL // NW
CHUNK = NT // L
NVEC = NT // L
RADIX = 256
HIST = RADIX * L
MININT = np.int32(-2147483648)


def _transpose_body(x_ref, o_ref):
    o_ref[...] = lax.bitcast_convert_type(x_ref[...].T, jnp.int32)


def _transpose(x):
    bt, bc = 1024, 512
    return pl.pallas_call(
        _transpose_body,
        grid=(NT // bt, NCOL // bc),
        in_specs=[pl.BlockSpec((bt, bc), lambda i, j: (i, j))],
        out_specs=pl.BlockSpec((bc, bt), lambda i, j: (j, i)),
        out_shape=jax.ShapeDtypeStruct((NCOL, NT), jnp.int32),
    )(x)


def _sc_body(pred_hbm, obs_hbm, out_hbm,
             a0, a1, a2, a3, b0, b1, b2, b3,
             hx0, hx1, hx2, hx3, hy0, hy1, hy2, hy3,
             tt0, tt1, tt2, tt3, dg0, dg1, dg2, dg3,
             stage, sem):
    cid = lax.axis_index("c")
    sid = lax.axis_index("s")
    wid = sid * NC + cid
    lane = lax.iota(jnp.int32, L)
    ones = jnp.ones((L,), jnp.int32)
    zi = jnp.zeros((L,), jnp.int32)
    fz = jnp.zeros((L,), jnp.float32)
    lane0 = lane == 0
    A = [a0, a1, a2, a3]
    B = [b0, b1, b2, b3]
    HX = [hx0, hx1, hx2, hx3]
    HY = [hy0, hy1, hy2, hy3]
    TOT = [tt0, tt1, tt2, tt3]
    DB = [dg0, dg1, dg2, dg3]

    def dmad(s, c0):
        ref = pred_hbm if s % 2 == 0 else obs_hbm
        return pltpu.make_async_copy(ref.at[c0 + s // 2], A[s], sem)

    def zinit(i, _):
        for s in range(4):
            HX[s][pl.ds(i * L, L)] = zi
        return 0

    lax.fori_loop(0, HIST // L, zinit, 0)

    base0 = wid * CPW
    for s in range(4):
        dmad(s, base0).start()

    def half_step(c0, cnext, HA, HB, accs):
        with jax.named_scope("dmawait"):
            for s in range(4):
                dmad(s, c0).wait()

        with jax.named_scope("h0"):
            @plsc.parallel_loop(0, NVEC, unroll=4)
            def h0(i):
                ys = [A[s][pl.ds(i * L, L)] for s in range(4)]
                for s in range(4):
                    m = ys[s] ^ (MININT | (ys[s] >> 31))
                    d = lax.shift_right_logical(m, 8) & 255
                    plsc.addupdate_scatter(HA[s], [(d << 4) | lane], ones)

        for t in range(3):
            cur = HA if t % 2 == 0 else HB
            nxt = HB if t % 2 == 0 else HA
            src = A if t % 2 == 0 else B
            dst = B if t % 2 == 0 else A

            with jax.named_scope("scan1"):
                @plsc.parallel_loop(0, RADIX, unroll=4)
                def l1(g):
                    vs = [cur[s][pl.ds(g * L, L)] for s in range(4)]
                    for s in range(4):
                        sc = plsc.cumsum(vs[s])
                        cur[s][pl.ds(g * L, L)] = sc - vs[s]
                        plsc.store_scatter(TOT[s], [g + zi], jnp.sum(vs[s]) + zi,
                                           mask=lane0)

            def l2(g, carry):
                nc = []
                for s in range(4):
                    v = TOT[s][pl.ds(g * L, L)]
                    sc = plsc.cumsum(v)
                    DB[s][pl.ds(g * L, L)] = (sc - v) + carry[s]
                    nc.append(carry[s] + jnp.sum(v))
                return tuple(nc)

            with jax.named_scope("scan2"):
                plsc.parallel_loop(0, RADIX // L, carry=(jnp.int32(0),) * 4)(l2)

            with jax.named_scope("scan3"):
                @plsc.parallel_loop(0, RADIX // L)
                def l3(gg):
                    dvecs = [DB[s][pl.ds(gg * L, L)] for s in range(4)]
                    for j in range(L):
                        exls = [cur[s][pl.ds(gg * RADIX + j * L, L)] for s in range(4)]
                        for s in range(4):
                            cur[s][pl.ds(gg * RADIX + j * L, L)] = exls[s] + dvecs[s][j]
                            nxt[s][pl.ds(gg * RADIX + j * L, L)] = zi

            def keys_at(i):
                ks = [src[s][pl.ds(i * L, L)] for s in range(4)]
                if t == 0:
                    outs = [k ^ (MININT | (k >> 31)) for k in ks]
                    digs = [lax.shift_right_logical(m, 8) & 255 for m in outs]
                elif t == 1:
                    outs = ks
                    digs = [lax.shift_right_logical(k, 16) & 255 for k in ks]
                else:
                    outs = ks
                    digs = [lax.shift_right_logical(k, 24) for k in ks]
                return tuple(outs), tuple([(d << 4) | lane for d in digs])

            def pb(i, carry):
                outs, hidxs = carry
                rs = [plsc.load_gather(cur[s], [hidxs[s]]) for s in range(4)]
                nxt_carry = keys_at(jnp.minimum(i + 1, NVEC - 1))
                for s in range(4):
                    plsc.store_scatter(cur[s], [hidxs[s]], rs[s] + ones)
                for s in range(4):
                    pos = ((rs[s] & (CHUNK - 1)) << 4) | (rs[s] >> 9)
                    plsc.store_scatter(dst[s], [pos], outs[s])
                return nxt_carry

            with jax.named_scope("permute"):
                lax.fori_loop(0, NVEC, pb, keys_at(0), unroll=8)

            if t < 2:
                sh2 = 16 if t == 0 else 24

                with jax.named_scope("hnx"):
                    @plsc.parallel_loop(0, NVEC, unroll=4)
                    def hnx(i):
                        ks = [dst[s][pl.ds(i * L, L)] for s in range(4)]
                        for s in range(4):
                            d = lax.shift_right_logical(ks[s], sh2)
                            if sh2 < 24:
                                d = d & 255
                            plsc.addupdate_scatter(nxt[s], [(d << 4) | lane], ones)

        for s in range(4):
            dmad(s, cnext).start()

        def dbody(i, acc):
            ys = [B[s][pl.ds((2 * i + par) * L, L)]
                  for par in range(2) for s in range(4)]
            fs = []
            for y in ys:
                x = y ^ (MININT | ~(y >> 31))
                fs.append(lax.bitcast_convert_type(x, jnp.float32))
            na = list(acc)
            for par in range(2):
                for pi in range(2):
                    fa = fs[par * 4 + 2 * pi]
                    fb = fs[par * 4 + 2 * pi + 1]
                    na[2 * pi + par] = na[2 * pi + par] + jnp.abs(fa - fb)
            return tuple(na)

        with jax.named_scope("diff"):
            return plsc.parallel_loop(0, NVEC // 2, carry=accs)(dbody)

    def outer(q, accs):
        c0 = base0 + 4 * q
        accs = half_step(c0, c0 + 2, HX, HY, accs)
        cnext = jnp.minimum(c0 + 4, NCOL - 2)
        accs = half_step(c0 + 2, cnext, HY, HX, accs)
        return accs

    accs = lax.fori_loop(0, CPW // 4, outer, (fz, fz, fz, fz))
    for s in range(4):
        dmad(s, 0).wait()
    stage[...] = (accs[0] + accs[1]) + (accs[2] + accs[3])
    pltpu.sync_copy(stage, out_hbm.at[wid])


_sc_wasserstein = functools.partial(
    pl.kernel,
    out_type=jax.ShapeDtypeStruct((NW, L), jnp.float32),
    mesh=plsc.VectorSubcoreMesh(core_axis_name="c", subcore_axis_name="s"),
    compiler_params=pltpu.CompilerParams(needs_layout_passes=False),
    scratch_types=(
        [pltpu.VMEM((NT,), jnp.int32) for _ in range(8)]
        + [pltpu.VMEM((HIST,), jnp.int32) for _ in range(8)]
        + [pltpu.VMEM((RADIX,), jnp.int32) for _ in range(8)]
        + [pltpu.VMEM((L,), jnp.float32)]
        + [pltpu.SemaphoreType.DMA]
    ),
)(_sc_body)


def kernel(pred_waveforms, obs_waveforms):
    nt, ntr, ch = pred_waveforms.shape
    pred_t = lax.bitcast_convert_type(
        jnp.transpose(pred_waveforms.reshape(nt, ntr * ch)), jnp.int32)
    obs_t = lax.bitcast_convert_type(
        jnp.transpose(obs_waveforms.reshape(nt, ntr * ch)), jnp.int32)
    partials = _sc_wasserstein(pred_t, obs_t)
    return jnp.sum(partials) / (nt * ntr * ch)
